# baseline (speedup 1.0000x reference)
import jax
import jax.numpy as jnp
from jax import lax
from jax.experimental import pallas as pl
from jax.experimental.pallas import tpu as pltpu
from jax.experimental.pallas import tpu_sc as plsc

_B = 4096
_D = 32
_W = 128
_NC, _NS = 2, 16
_NW = _NC * _NS
_BPW = _B // _NW


def _sc_gather_body(hi2, tab, out, idx_v, buf, sem):
    wid = lax.axis_index("s") * _NC + lax.axis_index("c")
    base = wid * _BPW
    pltpu.sync_copy(hi2.at[wid], idx_v)
    pltpu.async_copy(tab.at[idx_v], buf, sem).wait()
    pltpu.sync_copy(buf, out.at[pl.ds(base, _BPW)])


@jax.jit
def _sc_gather_one(hi2, tab):
    mesh = plsc.VectorSubcoreMesh(
        core_axis_name="c", subcore_axis_name="s",
        num_cores=_NC, num_subcores=_NS)
    return pl.kernel(
        _sc_gather_body,
        out_type=jax.ShapeDtypeStruct((_B, _W), jnp.float32),
        mesh=mesh,
        scratch_types=[
            pltpu.VMEM((_BPW,), jnp.int32),
            pltpu.VMEM((_BPW, _W), jnp.float32),
            pltpu.SemaphoreType.DMA,
        ],
        compiler_params=pltpu.CompilerParams(use_tc_tiling_on_sc=True),
    )(hi2, tab)


def _tc_mlp_body(bmfu_ref, bmfi_ref, bmlu_ref, bmli_ref, ulo_ref, ilo_ref,
                 sel_ref, w1u_ref, w1i_ref, b1_ref, w2_ref, b2_ref,
                 w3_ref, b3_ref, wp_mf_ref, wp_mlp_ref, bp_ref, out_ref):
    blk = lax.broadcasted_iota(jnp.int32, (_B, _W), 1) >> 5
    mu = blk == ulo_ref[...]
    mi = blk == ilo_ref[...]
    zero = jnp.zeros((), jnp.float32)
    dot = lambda a, b: jnp.dot(a, b, preferred_element_type=jnp.float32)
    sel = sel_ref[...]
    mf = (dot(jnp.where(mu, bmfu_ref[...], zero), sel)
          * dot(jnp.where(mi, bmfi_ref[...], zero), sel))
    h = jnp.maximum(
        dot(jnp.where(mu, bmlu_ref[...], zero), w1u_ref[...])
        + dot(jnp.where(mi, bmli_ref[...], zero), w1i_ref[...])
        + b1_ref[...][None, :], 0.0)
    h = jnp.maximum(dot(h, w2_ref[...]) + b2_ref[...][None, :], 0.0)
    h = jnp.maximum(dot(h, w3_ref[...]) + b3_ref[...][None, :], 0.0)
    logit = (jnp.sum(mf * wp_mf_ref[...][None, :], axis=1, keepdims=True)
             + jnp.sum(h * wp_mlp_ref[...][None, :], axis=1, keepdims=True)
             + bp_ref[...][None, :])
    out_ref[...] = jax.nn.sigmoid(logit)


@jax.jit
def _tc_mlp(bmfu, bmfi, bmlu, bmli, ulo, ilo,
            sel, w1u, w1i, b1, w2, b2, w3, b3, wp_mf, wp_mlp, bp):
    return pl.pallas_call(
        _tc_mlp_body,
        out_shape=jax.ShapeDtypeStruct((_B, 1), jnp.float32),
    )(bmfu, bmfi, bmlu, bmli, ulo, ilo,
      sel, w1u, w1i, b1, w2, b2, w3, b3, wp_mf, wp_mlp, bp)


def kernel(user_ids, item_ids, mf_user_table, mf_item_table,
           mlp_user_table, mlp_item_table, W1, b1, W2, b2, W3, b3, Wp, bp):
    uids = user_ids.astype(jnp.int32)
    iids = item_ids.astype(jnp.int32)
    uhi = (uids >> 2).reshape(_NW, _BPW)
    ihi = (iids >> 2).reshape(_NW, _BPW)
    ulo = (uids & 3).reshape(_B, 1)
    ilo = (iids & 3).reshape(_B, 1)
    bmfu = _sc_gather_one(uhi, mf_user_table.reshape(-1, _W))
    bmfi = _sc_gather_one(ihi, mf_item_table.reshape(-1, _W))
    bmlu = _sc_gather_one(uhi, mlp_user_table.reshape(-1, _W))
    bmli = _sc_gather_one(ihi, mlp_item_table.reshape(-1, _W))
    sel = jnp.tile(jnp.eye(_D, dtype=jnp.float32), (_W // _D, 1))
    w1u = jnp.tile(W1[:_D, :], (_W // _D, 1))
    w1i = jnp.tile(W1[_D:, :], (_W // _D, 1))
    return _tc_mlp(
        bmfu, bmfi, bmlu, bmli, ulo, ilo,
        sel, w1u, w1i, b1, W2, b2, W3, b3,
        Wp[:_D, 0], Wp[_D:, 0], bp)
